# baseline (speedup 1.0000x reference)
import functools

import jax
import jax.numpy as jnp
from jax import lax
from jax.experimental import pallas as pl
from jax.experimental.pallas import tpu as pltpu
from jax.experimental.pallas import tpu_sc as plsc

N = 10000
E = 640000
FEA = 67
K = 3
H1 = 16
OUT = 2

NP = 10240
NW = 32
CH = 128
SLAB = 2
NB = 4
NCH = 158
TS = NCH // SLAB
NG = TS // NB
NT = TS - NG * NB
EPT = NCH * CH
E2 = NW * EPT
STRIPE = NP // 16

F1 = 48
F2 = 16

BLK = 1024
GRID = (NP // BLK,)

_f32 = jnp.float32


def _mesh():
    return plsc.VectorSubcoreMesh(core_axis_name="c", subcore_axis_name="s")


@functools.partial(
    pl.kernel,
    mesh=_mesh(),
    out_type=jax.ShapeDtypeStruct((NW, NP), _f32),
    scratch_types=[
        pltpu.VMEM((EPT,), jnp.int32),
        pltpu.VMEM((NP,), _f32),
    ],
    compiler_params=pltpu.CompilerParams(needs_layout_passes=False),
)
def _sc_degree(dst_flat, zeros_np, deg_out, idx_v, deg_v):
    wid = lax.axis_index("s") * 2 + lax.axis_index("c")
    pltpu.sync_copy(dst_flat.at[wid], idx_v)
    pltpu.sync_copy(zeros_np, deg_v)
    ones = jnp.ones((16,), _f32)

    def body(i, carry):
        dvec = idx_v[pl.ds(i * 16, 16)]
        plsc.addupdate_scatter(deg_v, [dvec], ones)
        return carry

    lax.fori_loop(0, EPT // 16, body, 0)
    pltpu.sync_copy(deg_v, deg_out.at[wid])


def _make_prop(F):
    @functools.partial(
        pl.kernel,
        mesh=_mesh(),
        out_type=jax.ShapeDtypeStruct((2, NP, F), _f32),
        scratch_types=(
            [pltpu.VMEM((NCH // SLAB, SLAB * CH), jnp.int32)] * 2
            + [pltpu.VMEM((SLAB * CH, F), _f32)] * NB
            + [pltpu.VMEM_SHARED((NP, F), _f32)]
            + [pltpu.SemaphoreType.DMA] * NB
        ),
        compiler_params=pltpu.CompilerParams(
            needs_layout_passes=False, use_tc_tiling_on_sc=False),
    )
    def prop(ys, src3, dst3, zeros_npf, out, isrc, idst, *rest):
        bufs = rest[:NB]
        acc = rest[NB]
        gsems = rest[NB + 1:2 * NB + 1]
        c = lax.axis_index("c")
        s = lax.axis_index("s")
        wid = s * 2 + c
        pltpu.sync_copy(zeros_npf.at[pl.ds(s * STRIPE, STRIPE)],
                        acc.at[pl.ds(s * STRIPE, STRIPE)])
        pltpu.sync_copy(src3.at[wid], isrc)
        pltpu.sync_copy(dst3.at[wid], idst)
        plsc.subcore_barrier()

        for b in range(NB):
            pltpu.async_copy(ys.at[isrc.at[b]], bufs[b], gsems[b])

        def body(g, carry):
            base = g * NB
            for b in range(NB):
                t = base + b
                pltpu.make_async_copy(ys.at[isrc.at[t]], bufs[b],
                                      gsems[b]).wait()
                pltpu.sync_copy(bufs[b], acc.at[idst.at[t]], add=True)

                @pl.when(t + NB < TS)
                def _():
                    pltpu.async_copy(ys.at[isrc.at[t + NB]], bufs[b],
                                     gsems[b])
            return carry

        lax.fori_loop(0, NG, body, 0)
        for b in range(NT):
            t = NG * NB + b
            pltpu.make_async_copy(ys.at[isrc.at[t]], bufs[b],
                                  gsems[b]).wait()
            pltpu.sync_copy(bufs[b], acc.at[idst.at[t]], add=True)
        plsc.subcore_barrier()
        pltpu.sync_copy(acc.at[pl.ds(s * STRIPE, STRIPE)],
                        out.at[c, pl.ds(s * STRIPE, STRIPE)])

    return prop


_sc_prop48 = _make_prop(F1)
_sc_prop16 = _make_prop(F2)


def _tc1_body(degp_ref, x_ref, w1r_ref, b1_ref, w1i_ref,
              dinv_ref, root1_ref, y0s_ref):
    deg = jnp.sum(degp_ref[...], axis=0)
    dinv = jnp.where(deg > 0, lax.rsqrt(deg), 0.0)[:, None]
    dinv_ref[...] = dinv
    xb = x_ref[...]
    root1_ref[...] = (
        jnp.dot(xb, w1r_ref[...], preferred_element_type=_f32) + b1_ref[...])
    y0s_ref[...] = dinv * jnp.dot(xb, w1i_ref[...], preferred_element_type=_f32)


_tc1 = pl.pallas_call(
    _tc1_body,
    grid=GRID,
    in_specs=[
        pl.BlockSpec((NW, BLK), lambda i: (0, i)),
        pl.BlockSpec((BLK, 128), lambda i: (i, 0)),
        pl.BlockSpec((128, F1), lambda i: (0, 0)),
        pl.BlockSpec((1, F1), lambda i: (0, 0)),
        pl.BlockSpec((128, F1), lambda i: (0, 0)),
    ],
    out_specs=[
        pl.BlockSpec((BLK, 1), lambda i: (i, 0)),
        pl.BlockSpec((BLK, F1), lambda i: (i, 0)),
        pl.BlockSpec((BLK, F1), lambda i: (i, 0)),
    ],
    out_shape=[
        jax.ShapeDtypeStruct((NP, 1), _f32),
        jax.ShapeDtypeStruct((NP, F1), _f32),
        jax.ShapeDtypeStruct((NP, F1), _f32),
    ],
)


def _tc2_body(agg_ref, dinv_ref, root1_ref, wh_ref, y1s_ref):
    a = agg_ref[0] + agg_ref[1]
    out0 = jnp.maximum(dinv_ref[...] * a + root1_ref[...], 0.0)
    y1s_ref[...] = dinv_ref[...] * jnp.dot(
        out0, wh_ref[...], preferred_element_type=_f32)


_tc2 = pl.pallas_call(
    _tc2_body,
    grid=GRID,
    in_specs=[
        pl.BlockSpec((2, BLK, F1), lambda i: (0, i, 0)),
        pl.BlockSpec((BLK, 1), lambda i: (i, 0)),
        pl.BlockSpec((BLK, F1), lambda i: (i, 0)),
        pl.BlockSpec((F1, F1), lambda i: (0, 0)),
    ],
    out_specs=pl.BlockSpec((BLK, F1), lambda i: (i, 0)),
    out_shape=jax.ShapeDtypeStruct((NP, F1), _f32),
)


def _tc3_body(agg_ref, dinv_ref, root1_ref, w2r_ref, b2_ref, w2i_ref,
              root2_ref, z0s_ref):
    a = agg_ref[0] + agg_ref[1]
    out1 = jnp.maximum(dinv_ref[...] * a + root1_ref[...], 0.0)
    h = (out1[:, 0:16] + out1[:, 16:32] + out1[:, 32:48]) * (1.0 / 3.0)
    h = jnp.maximum(h, 0.0)
    root2_ref[...] = (
        jnp.dot(h, w2r_ref[...], preferred_element_type=_f32) + b2_ref[...])
    z0s_ref[...] = dinv_ref[...] * jnp.dot(
        h, w2i_ref[...], preferred_element_type=_f32)


_tc3 = pl.pallas_call(
    _tc3_body,
    grid=GRID,
    in_specs=[
        pl.BlockSpec((2, BLK, F1), lambda i: (0, i, 0)),
        pl.BlockSpec((BLK, 1), lambda i: (i, 0)),
        pl.BlockSpec((BLK, F1), lambda i: (i, 0)),
        pl.BlockSpec((H1, F2), lambda i: (0, 0)),
        pl.BlockSpec((1, F2), lambda i: (0, 0)),
        pl.BlockSpec((H1, F2), lambda i: (0, 0)),
    ],
    out_specs=[
        pl.BlockSpec((BLK, F2), lambda i: (i, 0)),
        pl.BlockSpec((BLK, F2), lambda i: (i, 0)),
    ],
    out_shape=[
        jax.ShapeDtypeStruct((NP, F2), _f32),
        jax.ShapeDtypeStruct((NP, F2), _f32),
    ],
)


def _tc4_body(agg_ref, dinv_ref, root2_ref, wh_ref, z1s_ref):
    a = agg_ref[0] + agg_ref[1]
    out2 = dinv_ref[...] * a + root2_ref[...]
    z1s_ref[...] = dinv_ref[...] * jnp.dot(
        out2, wh_ref[...], preferred_element_type=_f32)


_tc4 = pl.pallas_call(
    _tc4_body,
    grid=GRID,
    in_specs=[
        pl.BlockSpec((2, BLK, F2), lambda i: (0, i, 0)),
        pl.BlockSpec((BLK, 1), lambda i: (i, 0)),
        pl.BlockSpec((BLK, F2), lambda i: (i, 0)),
        pl.BlockSpec((F2, F2), lambda i: (0, 0)),
    ],
    out_specs=pl.BlockSpec((BLK, F2), lambda i: (i, 0)),
    out_shape=jax.ShapeDtypeStruct((NP, F2), _f32),
)


def _tc5_body(agg_ref, dinv_ref, root2_ref, out_ref):
    a = agg_ref[0] + agg_ref[1]
    o3 = dinv_ref[...] * a + root2_ref[...]
    o = (o3[:, 0:2] + o3[:, 2:4] + o3[:, 4:6]) * (1.0 / 3.0)
    m = jnp.max(o, axis=1, keepdims=True)
    lse = m + jnp.log(jnp.sum(jnp.exp(o - m), axis=1, keepdims=True))
    out_ref[...] = o - lse


_tc5 = pl.pallas_call(
    _tc5_body,
    grid=GRID,
    in_specs=[
        pl.BlockSpec((2, BLK, F2), lambda i: (0, i, 0)),
        pl.BlockSpec((BLK, 1), lambda i: (i, 0)),
        pl.BlockSpec((BLK, F2), lambda i: (i, 0)),
    ],
    out_specs=pl.BlockSpec((BLK, OUT), lambda i: (i, 0)),
    out_shape=jax.ShapeDtypeStruct((NP, OUT), _f32),
)


def kernel(x, edge_index, w1_init, w1_hop, w1_root, b1,
           w2_init, w2_hop, w2_root, b2):
    x_pad = jnp.zeros((NP, 128), _f32).at[:N, :FEA].set(x)
    padi = jnp.full((E2 - E,), N, jnp.int32)
    src_f = jnp.concatenate([edge_index[0], padi]).reshape(NW, EPT)
    dst_f = jnp.concatenate([edge_index[1], padi]).reshape(NW, EPT)
    src3 = src_f.reshape(NW, NCH // SLAB, SLAB * CH)
    dst3 = dst_f.reshape(NW, NCH // SLAB, SLAB * CH)

    w1r = jnp.zeros((128, F1), _f32).at[:FEA].set(
        w1_root.transpose(1, 0, 2).reshape(FEA, F1))
    w1i = jnp.zeros((128, F1), _f32).at[:FEA].set(
        w1_init.transpose(1, 0, 2).reshape(FEA, F1))
    b1f = b1.reshape(1, F1)
    w1h = jax.scipy.linalg.block_diag(w1_hop[0], w1_hop[1], w1_hop[2])

    kout = K * OUT
    w2r = jnp.zeros((H1, F2), _f32).at[:, :kout].set(
        w2_root.transpose(1, 0, 2).reshape(H1, kout))
    w2i = jnp.zeros((H1, F2), _f32).at[:, :kout].set(
        w2_init.transpose(1, 0, 2).reshape(H1, kout))
    b2f = jnp.zeros((1, F2), _f32).at[:, :kout].set(b2.reshape(1, kout))
    w2h = jnp.zeros((F2, F2), _f32).at[:kout, :kout].set(
        jax.scipy.linalg.block_diag(w2_hop[0], w2_hop[1], w2_hop[2]))

    zeros_np = jnp.zeros((NP,), _f32)
    zeros48 = jnp.zeros((NP, F1), _f32)
    zeros16 = jnp.zeros((NP, F2), _f32)

    deg_parts = _sc_degree(dst_f, zeros_np)
    dinv, root1, y0s = _tc1(deg_parts, x_pad, w1r, b1f, w1i)
    agg0 = _sc_prop48(y0s, src3, dst3, zeros48)
    y1s = _tc2(agg0, dinv, root1, w1h)
    agg1 = _sc_prop48(y1s, src3, dst3, zeros48)
    root2, z0s = _tc3(agg1, dinv, root1, w2r, b2f, w2i)
    agg2 = _sc_prop16(z0s, src3, dst3, zeros16)
    z1s = _tc4(agg2, dinv, root2, w2h)
    agg3 = _sc_prop16(z1s, src3, dst3, zeros16)
    out = _tc5(agg3, dinv, root2)
    return out[:N]


# speedup vs baseline: 1.0119x; 1.0119x over previous
import functools

import jax
import jax.numpy as jnp
from jax import lax
from jax.experimental import pallas as pl
from jax.experimental.pallas import tpu as pltpu
from jax.experimental.pallas import tpu_sc as plsc

N = 10000
E = 640000
FEA = 67
K = 3
H1 = 16
OUT = 2

NP = 10240
NW = 32
CH = 128
SLAB = 1
NB = 8
NCH = 158
TS = NCH // SLAB
NG = TS // NB
NT = TS - NG * NB
EPT = NCH * CH
E2 = NW * EPT
STRIPE = NP // 16

F1 = 48
F2 = 16

BLK = NP
GRID = (NP // BLK,)

_f32 = jnp.float32


def _mesh():
    return plsc.VectorSubcoreMesh(core_axis_name="c", subcore_axis_name="s")


@functools.partial(
    pl.kernel,
    mesh=_mesh(),
    out_type=jax.ShapeDtypeStruct((NW, NP), _f32),
    scratch_types=[
        pltpu.VMEM((EPT,), jnp.int32),
        pltpu.VMEM((NP,), _f32),
    ],
    compiler_params=pltpu.CompilerParams(needs_layout_passes=False),
)
def _sc_degree(dst_flat, zeros_np, deg_out, idx_v, deg_v):
    wid = lax.axis_index("s") * 2 + lax.axis_index("c")
    pltpu.sync_copy(dst_flat.at[wid], idx_v)
    pltpu.sync_copy(zeros_np, deg_v)
    ones = jnp.ones((16,), _f32)

    def body(i, carry):
        dvec = idx_v[pl.ds(i * 16, 16)]
        plsc.addupdate_scatter(deg_v, [dvec], ones)
        return carry

    lax.fori_loop(0, EPT // 16, body, 0)
    pltpu.sync_copy(deg_v, deg_out.at[wid])


def _make_prop(F):
    @functools.partial(
        pl.kernel,
        mesh=_mesh(),
        out_type=jax.ShapeDtypeStruct((2, NP, F), _f32),
        scratch_types=(
            [pltpu.VMEM((NCH // SLAB, SLAB * CH), jnp.int32)] * 2
            + [pltpu.VMEM((SLAB * CH, F), _f32)] * NB
            + [pltpu.VMEM_SHARED((NP, F), _f32)]
            + [pltpu.SemaphoreType.DMA] * NB
        ),
        compiler_params=pltpu.CompilerParams(
            needs_layout_passes=False, use_tc_tiling_on_sc=False),
    )
    def prop(ys, src3, dst3, zeros_npf, out, isrc, idst, *rest):
        bufs = rest[:NB]
        acc = rest[NB]
        gsems = rest[NB + 1:2 * NB + 1]
        c = lax.axis_index("c")
        s = lax.axis_index("s")
        wid = s * 2 + c
        pltpu.sync_copy(zeros_npf.at[pl.ds(s * STRIPE, STRIPE)],
                        acc.at[pl.ds(s * STRIPE, STRIPE)])
        pltpu.sync_copy(src3.at[wid], isrc)
        pltpu.sync_copy(dst3.at[wid], idst)
        plsc.subcore_barrier()

        for b in range(NB):
            pltpu.async_copy(ys.at[isrc.at[b]], bufs[b], gsems[b])

        def body(g, carry):
            base = g * NB
            for b in range(NB):
                t = base + b
                pltpu.make_async_copy(ys.at[isrc.at[t]], bufs[b],
                                      gsems[b]).wait()
                pltpu.sync_copy(bufs[b], acc.at[idst.at[t]], add=True)

                @pl.when(t + NB < TS)
                def _():
                    pltpu.async_copy(ys.at[isrc.at[t + NB]], bufs[b],
                                     gsems[b])
            return carry

        lax.fori_loop(0, NG, body, 0)
        for b in range(NT):
            t = NG * NB + b
            pltpu.make_async_copy(ys.at[isrc.at[t]], bufs[b],
                                  gsems[b]).wait()
            pltpu.sync_copy(bufs[b], acc.at[idst.at[t]], add=True)
        plsc.subcore_barrier()
        pltpu.sync_copy(acc.at[pl.ds(s * STRIPE, STRIPE)],
                        out.at[c, pl.ds(s * STRIPE, STRIPE)])

    return prop


_sc_prop48 = _make_prop(F1)
_sc_prop16 = _make_prop(F2)


def _tc1_body(degp_ref, x_ref, w1r_ref, b1_ref, w1i_ref,
              dinv_ref, root1_ref, y0s_ref):
    deg = jnp.sum(degp_ref[...], axis=0)
    dinv = jnp.where(deg > 0, lax.rsqrt(deg), 0.0)[:, None]
    dinv_ref[...] = dinv
    xb = x_ref[...]
    root1_ref[...] = (
        jnp.dot(xb, w1r_ref[...], preferred_element_type=_f32) + b1_ref[...])
    y0s_ref[...] = dinv * jnp.dot(xb, w1i_ref[...], preferred_element_type=_f32)


_tc1 = pl.pallas_call(
    _tc1_body,
    grid=GRID,
    in_specs=[
        pl.BlockSpec((NW, BLK), lambda i: (0, i)),
        pl.BlockSpec((BLK, 128), lambda i: (i, 0)),
        pl.BlockSpec((128, F1), lambda i: (0, 0)),
        pl.BlockSpec((1, F1), lambda i: (0, 0)),
        pl.BlockSpec((128, F1), lambda i: (0, 0)),
    ],
    out_specs=[
        pl.BlockSpec((BLK, 1), lambda i: (i, 0)),
        pl.BlockSpec((BLK, F1), lambda i: (i, 0)),
        pl.BlockSpec((BLK, F1), lambda i: (i, 0)),
    ],
    out_shape=[
        jax.ShapeDtypeStruct((NP, 1), _f32),
        jax.ShapeDtypeStruct((NP, F1), _f32),
        jax.ShapeDtypeStruct((NP, F1), _f32),
    ],
)


def _tc2_body(agg_ref, dinv_ref, root1_ref, wh_ref, y1s_ref):
    a = agg_ref[0] + agg_ref[1]
    out0 = jnp.maximum(dinv_ref[...] * a + root1_ref[...], 0.0)
    y1s_ref[...] = dinv_ref[...] * jnp.dot(
        out0, wh_ref[...], preferred_element_type=_f32)


_tc2 = pl.pallas_call(
    _tc2_body,
    grid=GRID,
    in_specs=[
        pl.BlockSpec((2, BLK, F1), lambda i: (0, i, 0)),
        pl.BlockSpec((BLK, 1), lambda i: (i, 0)),
        pl.BlockSpec((BLK, F1), lambda i: (i, 0)),
        pl.BlockSpec((F1, F1), lambda i: (0, 0)),
    ],
    out_specs=pl.BlockSpec((BLK, F1), lambda i: (i, 0)),
    out_shape=jax.ShapeDtypeStruct((NP, F1), _f32),
)


def _tc3_body(agg_ref, dinv_ref, root1_ref, w2r_ref, b2_ref, w2i_ref,
              root2_ref, z0s_ref):
    a = agg_ref[0] + agg_ref[1]
    out1 = jnp.maximum(dinv_ref[...] * a + root1_ref[...], 0.0)
    h = (out1[:, 0:16] + out1[:, 16:32] + out1[:, 32:48]) * (1.0 / 3.0)
    h = jnp.maximum(h, 0.0)
    root2_ref[...] = (
        jnp.dot(h, w2r_ref[...], preferred_element_type=_f32) + b2_ref[...])
    z0s_ref[...] = dinv_ref[...] * jnp.dot(
        h, w2i_ref[...], preferred_element_type=_f32)


_tc3 = pl.pallas_call(
    _tc3_body,
    grid=GRID,
    in_specs=[
        pl.BlockSpec((2, BLK, F1), lambda i: (0, i, 0)),
        pl.BlockSpec((BLK, 1), lambda i: (i, 0)),
        pl.BlockSpec((BLK, F1), lambda i: (i, 0)),
        pl.BlockSpec((H1, F2), lambda i: (0, 0)),
        pl.BlockSpec((1, F2), lambda i: (0, 0)),
        pl.BlockSpec((H1, F2), lambda i: (0, 0)),
    ],
    out_specs=[
        pl.BlockSpec((BLK, F2), lambda i: (i, 0)),
        pl.BlockSpec((BLK, F2), lambda i: (i, 0)),
    ],
    out_shape=[
        jax.ShapeDtypeStruct((NP, F2), _f32),
        jax.ShapeDtypeStruct((NP, F2), _f32),
    ],
)


def _tc4_body(agg_ref, dinv_ref, root2_ref, wh_ref, z1s_ref):
    a = agg_ref[0] + agg_ref[1]
    out2 = dinv_ref[...] * a + root2_ref[...]
    z1s_ref[...] = dinv_ref[...] * jnp.dot(
        out2, wh_ref[...], preferred_element_type=_f32)


_tc4 = pl.pallas_call(
    _tc4_body,
    grid=GRID,
    in_specs=[
        pl.BlockSpec((2, BLK, F2), lambda i: (0, i, 0)),
        pl.BlockSpec((BLK, 1), lambda i: (i, 0)),
        pl.BlockSpec((BLK, F2), lambda i: (i, 0)),
        pl.BlockSpec((F2, F2), lambda i: (0, 0)),
    ],
    out_specs=pl.BlockSpec((BLK, F2), lambda i: (i, 0)),
    out_shape=jax.ShapeDtypeStruct((NP, F2), _f32),
)


def _tc5_body(agg_ref, dinv_ref, root2_ref, out_ref):
    a = agg_ref[0] + agg_ref[1]
    o3 = dinv_ref[...] * a + root2_ref[...]
    o = (o3[:, 0:2] + o3[:, 2:4] + o3[:, 4:6]) * (1.0 / 3.0)
    m = jnp.max(o, axis=1, keepdims=True)
    lse = m + jnp.log(jnp.sum(jnp.exp(o - m), axis=1, keepdims=True))
    out_ref[...] = o - lse


_tc5 = pl.pallas_call(
    _tc5_body,
    grid=GRID,
    in_specs=[
        pl.BlockSpec((2, BLK, F2), lambda i: (0, i, 0)),
        pl.BlockSpec((BLK, 1), lambda i: (i, 0)),
        pl.BlockSpec((BLK, F2), lambda i: (i, 0)),
    ],
    out_specs=pl.BlockSpec((BLK, OUT), lambda i: (i, 0)),
    out_shape=jax.ShapeDtypeStruct((NP, OUT), _f32),
)


def kernel(x, edge_index, w1_init, w1_hop, w1_root, b1,
           w2_init, w2_hop, w2_root, b2):
    x_pad = jnp.zeros((NP, 128), _f32).at[:N, :FEA].set(x)
    padi = jnp.full((E2 - E,), N, jnp.int32)
    src_f = jnp.concatenate([edge_index[0], padi]).reshape(NW, EPT)
    dst_f = jnp.concatenate([edge_index[1], padi]).reshape(NW, EPT)
    src3 = src_f.reshape(NW, NCH // SLAB, SLAB * CH)
    dst3 = dst_f.reshape(NW, NCH // SLAB, SLAB * CH)

    w1r = jnp.zeros((128, F1), _f32).at[:FEA].set(
        w1_root.transpose(1, 0, 2).reshape(FEA, F1))
    w1i = jnp.zeros((128, F1), _f32).at[:FEA].set(
        w1_init.transpose(1, 0, 2).reshape(FEA, F1))
    b1f = b1.reshape(1, F1)
    w1h = jax.scipy.linalg.block_diag(w1_hop[0], w1_hop[1], w1_hop[2])

    kout = K * OUT
    w2r = jnp.zeros((H1, F2), _f32).at[:, :kout].set(
        w2_root.transpose(1, 0, 2).reshape(H1, kout))
    w2i = jnp.zeros((H1, F2), _f32).at[:, :kout].set(
        w2_init.transpose(1, 0, 2).reshape(H1, kout))
    b2f = jnp.zeros((1, F2), _f32).at[:, :kout].set(b2.reshape(1, kout))
    w2h = jnp.zeros((F2, F2), _f32).at[:kout, :kout].set(
        jax.scipy.linalg.block_diag(w2_hop[0], w2_hop[1], w2_hop[2]))

    zeros_np = jnp.zeros((NP,), _f32)
    zeros48 = jnp.zeros((NP, F1), _f32)
    zeros16 = jnp.zeros((NP, F2), _f32)

    deg_parts = _sc_degree(dst_f, zeros_np)
    dinv, root1, y0s = _tc1(deg_parts, x_pad, w1r, b1f, w1i)
    agg0 = _sc_prop48(y0s, src3, dst3, zeros48)
    y1s = _tc2(agg0, dinv, root1, w1h)
    agg1 = _sc_prop48(y1s, src3, dst3, zeros48)
    root2, z0s = _tc3(agg1, dinv, root1, w2r, b2f, w2i)
    agg2 = _sc_prop16(z0s, src3, dst3, zeros16)
    z1s = _tc4(agg2, dinv, root2, w2h)
    agg3 = _sc_prop16(z1s, src3, dst3, zeros16)
    out = _tc5(agg3, dinv, root2)
    return out[:N]


# speedup vs baseline: 1.1312x; 1.1179x over previous
import functools

import jax
import jax.numpy as jnp
from jax import lax
from jax.experimental import pallas as pl
from jax.experimental.pallas import tpu as pltpu
from jax.experimental.pallas import tpu_sc as plsc

N = 10000
E = 640000
FEA = 67
K = 3
H1 = 16
OUT = 2

NP = 10240
NW = 32
CH = 128
SLAB = 1
NB = 8
NCH = 158
TS = NCH // SLAB
NG = TS // NB
NT = TS - NG * NB
EPT = NCH * CH
E2 = NW * EPT
STRIPE = NP // 16

F1 = 48
F2 = 16

BLK = NP
GRID = (NP // BLK,)

_f32 = jnp.float32


def _mesh():
    return plsc.VectorSubcoreMesh(core_axis_name="c", subcore_axis_name="s")


@functools.partial(
    pl.kernel,
    mesh=_mesh(),
    out_type=jax.ShapeDtypeStruct((NW, NP), _f32),
    scratch_types=[
        pltpu.VMEM((EPT,), jnp.int32),
        pltpu.VMEM((NP,), _f32),
    ],
    compiler_params=pltpu.CompilerParams(needs_layout_passes=False),
)
def _sc_degree(dst_flat, zeros_np, deg_out, idx_v, deg_v):
    wid = lax.axis_index("s") * 2 + lax.axis_index("c")
    pltpu.sync_copy(dst_flat.at[wid], idx_v)
    pltpu.sync_copy(zeros_np, deg_v)
    ones = jnp.ones((16,), _f32)

    def body(i, carry):
        dvec = idx_v[pl.ds(i * 16, 16)]
        plsc.addupdate_scatter(deg_v, [dvec], ones)
        return carry

    lax.fori_loop(0, EPT // 16, body, 0)
    pltpu.sync_copy(deg_v, deg_out.at[wid])


def _make_prop(F):
    @functools.partial(
        pl.kernel,
        mesh=_mesh(),
        out_type=jax.ShapeDtypeStruct((2, NP, F), _f32),
        scratch_types=(
            [pltpu.VMEM((NCH // SLAB, SLAB * CH), jnp.int32)] * 2
            + [pltpu.VMEM((SLAB * CH, F), _f32)] * NB
            + [pltpu.VMEM_SHARED((NP, F), _f32)]
            + [pltpu.SemaphoreType.DMA] * NB
        ),
        compiler_params=pltpu.CompilerParams(
            needs_layout_passes=False, use_tc_tiling_on_sc=False),
    )
    def prop(ys, src3, dst3, zeros_npf, out, isrc, idst, *rest):
        bufs = rest[:NB]
        acc = rest[NB]
        gsems = rest[NB + 1:2 * NB + 1]
        c = lax.axis_index("c")
        s = lax.axis_index("s")
        wid = s * 2 + c
        pltpu.sync_copy(zeros_npf.at[pl.ds(s * STRIPE, STRIPE)],
                        acc.at[pl.ds(s * STRIPE, STRIPE)])
        pltpu.sync_copy(src3.at[wid], isrc)
        pltpu.sync_copy(dst3.at[wid], idst)
        plsc.subcore_barrier()

        for b in range(NB):
            pltpu.async_copy(ys.at[isrc.at[b]], bufs[b], gsems[b])

        def body(g, carry):
            base = g * NB
            for b in range(NB):
                t = base + b
                pltpu.make_async_copy(ys.at[isrc.at[t]], bufs[b],
                                      gsems[b]).wait()
                pltpu.sync_copy(bufs[b], acc.at[idst.at[t]], add=True)

                @pl.when(t + NB < TS)
                def _():
                    pltpu.async_copy(ys.at[isrc.at[t + NB]], bufs[b],
                                     gsems[b])
            return carry

        lax.fori_loop(0, NG, body, 0)
        for b in range(NT):
            t = NG * NB + b
            pltpu.make_async_copy(ys.at[isrc.at[t]], bufs[b],
                                  gsems[b]).wait()
            pltpu.sync_copy(bufs[b], acc.at[idst.at[t]], add=True)
        plsc.subcore_barrier()
        pltpu.sync_copy(acc.at[pl.ds(s * STRIPE, STRIPE)],
                        out.at[c, pl.ds(s * STRIPE, STRIPE)])

    return prop


_sc_prop48 = _make_prop(F1)
_sc_prop16 = _make_prop(F2)


def _tc0_body(x_ref, w1r_ref, b1_ref, w1i_ref, root1_ref, xw1i_ref):
    xb = x_ref[...]
    root1_ref[...] = (
        jnp.dot(xb, w1r_ref[...], preferred_element_type=_f32) + b1_ref[...])
    xw1i_ref[...] = jnp.dot(xb, w1i_ref[...], preferred_element_type=_f32)


_tc0 = pl.pallas_call(
    _tc0_body,
    grid=GRID,
    in_specs=[
        pl.BlockSpec((BLK, 128), lambda i: (i, 0)),
        pl.BlockSpec((128, F1), lambda i: (0, 0)),
        pl.BlockSpec((1, F1), lambda i: (0, 0)),
        pl.BlockSpec((128, F1), lambda i: (0, 0)),
    ],
    out_specs=[
        pl.BlockSpec((BLK, F1), lambda i: (i, 0)),
        pl.BlockSpec((BLK, F1), lambda i: (i, 0)),
    ],
    out_shape=[
        jax.ShapeDtypeStruct((NP, F1), _f32),
        jax.ShapeDtypeStruct((NP, F1), _f32),
    ],
)


def _tc1_body(degp_ref, xw1i_ref, dinv_ref, y0s_ref):
    deg = jnp.sum(degp_ref[...], axis=0)
    dinv = jnp.where(deg > 0, lax.rsqrt(deg), 0.0)[:, None]
    dinv_ref[...] = dinv
    y0s_ref[...] = dinv * xw1i_ref[...]


_tc1 = pl.pallas_call(
    _tc1_body,
    grid=GRID,
    in_specs=[
        pl.BlockSpec((NW, BLK), lambda i: (0, i)),
        pl.BlockSpec((BLK, F1), lambda i: (i, 0)),
    ],
    out_specs=[
        pl.BlockSpec((BLK, 1), lambda i: (i, 0)),
        pl.BlockSpec((BLK, F1), lambda i: (i, 0)),
    ],
    out_shape=[
        jax.ShapeDtypeStruct((NP, 1), _f32),
        jax.ShapeDtypeStruct((NP, F1), _f32),
    ],
)


def _tc2_body(agg_ref, dinv_ref, root1_ref, wh_ref, y1s_ref):
    a = agg_ref[0] + agg_ref[1]
    out0 = jnp.maximum(dinv_ref[...] * a + root1_ref[...], 0.0)
    y1s_ref[...] = dinv_ref[...] * jnp.dot(
        out0, wh_ref[...], preferred_element_type=_f32)


_tc2 = pl.pallas_call(
    _tc2_body,
    grid=GRID,
    in_specs=[
        pl.BlockSpec((2, BLK, F1), lambda i: (0, i, 0)),
        pl.BlockSpec((BLK, 1), lambda i: (i, 0)),
        pl.BlockSpec((BLK, F1), lambda i: (i, 0)),
        pl.BlockSpec((F1, F1), lambda i: (0, 0)),
    ],
    out_specs=pl.BlockSpec((BLK, F1), lambda i: (i, 0)),
    out_shape=jax.ShapeDtypeStruct((NP, F1), _f32),
)


def _tc3_body(agg_ref, dinv_ref, root1_ref, w2r_ref, b2_ref, w2i_ref,
              root2_ref, z0s_ref):
    a = agg_ref[0] + agg_ref[1]
    out1 = jnp.maximum(dinv_ref[...] * a + root1_ref[...], 0.0)
    h = (out1[:, 0:16] + out1[:, 16:32] + out1[:, 32:48]) * (1.0 / 3.0)
    h = jnp.maximum(h, 0.0)
    root2_ref[...] = (
        jnp.dot(h, w2r_ref[...], preferred_element_type=_f32) + b2_ref[...])
    z0s_ref[...] = dinv_ref[...] * jnp.dot(
        h, w2i_ref[...], preferred_element_type=_f32)


_tc3 = pl.pallas_call(
    _tc3_body,
    grid=GRID,
    in_specs=[
        pl.BlockSpec((2, BLK, F1), lambda i: (0, i, 0)),
        pl.BlockSpec((BLK, 1), lambda i: (i, 0)),
        pl.BlockSpec((BLK, F1), lambda i: (i, 0)),
        pl.BlockSpec((H1, F2), lambda i: (0, 0)),
        pl.BlockSpec((1, F2), lambda i: (0, 0)),
        pl.BlockSpec((H1, F2), lambda i: (0, 0)),
    ],
    out_specs=[
        pl.BlockSpec((BLK, F2), lambda i: (i, 0)),
        pl.BlockSpec((BLK, F2), lambda i: (i, 0)),
    ],
    out_shape=[
        jax.ShapeDtypeStruct((NP, F2), _f32),
        jax.ShapeDtypeStruct((NP, F2), _f32),
    ],
)


def _tc4_body(agg_ref, dinv_ref, root2_ref, wh_ref, z1s_ref):
    a = agg_ref[0] + agg_ref[1]
    out2 = dinv_ref[...] * a + root2_ref[...]
    z1s_ref[...] = dinv_ref[...] * jnp.dot(
        out2, wh_ref[...], preferred_element_type=_f32)


_tc4 = pl.pallas_call(
    _tc4_body,
    grid=GRID,
    in_specs=[
        pl.BlockSpec((2, BLK, F2), lambda i: (0, i, 0)),
        pl.BlockSpec((BLK, 1), lambda i: (i, 0)),
        pl.BlockSpec((BLK, F2), lambda i: (i, 0)),
        pl.BlockSpec((F2, F2), lambda i: (0, 0)),
    ],
    out_specs=pl.BlockSpec((BLK, F2), lambda i: (i, 0)),
    out_shape=jax.ShapeDtypeStruct((NP, F2), _f32),
)


def _tc5_body(agg_ref, dinv_ref, root2_ref, out_ref):
    a = agg_ref[0] + agg_ref[1]
    o3 = dinv_ref[...] * a + root2_ref[...]
    o = (o3[:, 0:2] + o3[:, 2:4] + o3[:, 4:6]) * (1.0 / 3.0)
    m = jnp.max(o, axis=1, keepdims=True)
    lse = m + jnp.log(jnp.sum(jnp.exp(o - m), axis=1, keepdims=True))
    out_ref[...] = o - lse


_tc5 = pl.pallas_call(
    _tc5_body,
    grid=GRID,
    in_specs=[
        pl.BlockSpec((2, BLK, F2), lambda i: (0, i, 0)),
        pl.BlockSpec((BLK, 1), lambda i: (i, 0)),
        pl.BlockSpec((BLK, F2), lambda i: (i, 0)),
    ],
    out_specs=pl.BlockSpec((BLK, OUT), lambda i: (i, 0)),
    out_shape=jax.ShapeDtypeStruct((NP, OUT), _f32),
)


def kernel(x, edge_index, w1_init, w1_hop, w1_root, b1,
           w2_init, w2_hop, w2_root, b2):
    x_pad = jnp.zeros((NP, 128), _f32).at[:N, :FEA].set(x)
    padi = jnp.full((E2 - E,), N, jnp.int32)
    src_f = jnp.concatenate([edge_index[0], padi]).reshape(NW, EPT)
    dst_f = jnp.concatenate([edge_index[1], padi]).reshape(NW, EPT)
    src3 = src_f.reshape(NW, NCH // SLAB, SLAB * CH)
    dst3 = dst_f.reshape(NW, NCH // SLAB, SLAB * CH)

    w1r = jnp.zeros((128, F1), _f32).at[:FEA].set(
        w1_root.transpose(1, 0, 2).reshape(FEA, F1))
    w1i = jnp.zeros((128, F1), _f32).at[:FEA].set(
        w1_init.transpose(1, 0, 2).reshape(FEA, F1))
    b1f = b1.reshape(1, F1)
    w1h = jax.scipy.linalg.block_diag(w1_hop[0], w1_hop[1], w1_hop[2])

    kout = K * OUT
    w2r = jnp.zeros((H1, F2), _f32).at[:, :kout].set(
        w2_root.transpose(1, 0, 2).reshape(H1, kout))
    w2i = jnp.zeros((H1, F2), _f32).at[:, :kout].set(
        w2_init.transpose(1, 0, 2).reshape(H1, kout))
    b2f = jnp.zeros((1, F2), _f32).at[:, :kout].set(b2.reshape(1, kout))
    w2h = jnp.zeros((F2, F2), _f32).at[:kout, :kout].set(
        jax.scipy.linalg.block_diag(w2_hop[0], w2_hop[1], w2_hop[2]))

    zeros_np = jnp.zeros((NP,), _f32)
    zeros48 = jnp.zeros((NP, F1), _f32)
    zeros16 = jnp.zeros((NP, F2), _f32)

    deg_parts = _sc_degree(dst_f, zeros_np)
    root1, xw1i = _tc0(x_pad, w1r, b1f, w1i)
    dinv, y0s = _tc1(deg_parts, xw1i)
    agg0 = _sc_prop48(y0s, src3, dst3, zeros48)
    y1s = _tc2(agg0, dinv, root1, w1h)
    agg1 = _sc_prop48(y1s, src3, dst3, zeros48)
    root2, z0s = _tc3(agg1, dinv, root1, w2r, b2f, w2i)
    agg2 = _sc_prop16(z0s, src3, dst3, zeros16)
    z1s = _tc4(agg2, dinv, root2, w2h)
    agg3 = _sc_prop16(z1s, src3, dst3, zeros16)
    out = _tc5(agg3, dinv, root2)
    return out[:N]


# speedup vs baseline: 1.5265x; 1.3494x over previous
import functools

import jax
import jax.numpy as jnp
from jax import lax
from jax.experimental import pallas as pl
from jax.experimental.pallas import tpu as pltpu
from jax.experimental.pallas import tpu_sc as plsc

N = 10000
E = 640000
FEA = 67
K = 3
H1 = 16
OUT = 2

NP = 10240
NW = 32
CH = 128
SLAB = 1
NB = 8
NCH = 157
TS = NCH // SLAB
NG = TS // NB
NT = TS - NG * NB
EPT = NCH * CH
E2 = NW * EPT
STRIPE = NP // 16

F1 = 48
F2 = 16

BLK = NP
GRID = (NP // BLK,)

_f32 = jnp.float32


def _mesh():
    return plsc.VectorSubcoreMesh(core_axis_name="c", subcore_axis_name="s")


@functools.partial(
    pl.kernel,
    mesh=_mesh(),
    out_type=jax.ShapeDtypeStruct((NW, NP), _f32),
    scratch_types=[
        pltpu.VMEM((EPT,), jnp.int32),
        pltpu.VMEM((NP,), _f32),
    ],
    compiler_params=pltpu.CompilerParams(needs_layout_passes=False),
)
def _sc_degree(dst_flat, zeros_np, deg_out, idx_v, deg_v):
    wid = lax.axis_index("s") * 2 + lax.axis_index("c")
    pltpu.sync_copy(dst_flat.at[wid], idx_v)
    pltpu.sync_copy(zeros_np, deg_v)
    ones = jnp.ones((16,), _f32)

    def body(i, carry):
        dvec = idx_v[pl.ds(i * 16, 16)]
        plsc.addupdate_scatter(deg_v, [dvec], ones)
        return carry

    lax.fori_loop(0, EPT // 16, body, 0)
    pltpu.sync_copy(deg_v, deg_out.at[wid])


def _make_prop(F):
    @functools.partial(
        pl.kernel,
        mesh=_mesh(),
        out_type=jax.ShapeDtypeStruct((2, NP, F), _f32),
        scratch_types=(
            [pltpu.VMEM((NCH // SLAB, SLAB * CH), jnp.int32)] * 2
            + [pltpu.VMEM((SLAB * CH, F), _f32)] * NB
            + [pltpu.VMEM_SHARED((NP, F), _f32)]
            + [pltpu.SemaphoreType.DMA] * NB
        ),
        compiler_params=pltpu.CompilerParams(
            needs_layout_passes=False, use_tc_tiling_on_sc=False),
    )
    def prop(ys, src3, dst3, zeros_npf, out, isrc, idst, *rest):
        bufs = rest[:NB]
        acc = rest[NB]
        gsems = rest[NB + 1:2 * NB + 1]
        c = lax.axis_index("c")
        s = lax.axis_index("s")
        wid = s * 2 + c
        pltpu.sync_copy(zeros_npf.at[pl.ds(s * STRIPE, STRIPE)],
                        acc.at[pl.ds(s * STRIPE, STRIPE)])
        pltpu.sync_copy(src3.at[wid], isrc)
        pltpu.sync_copy(dst3.at[wid], idst)
        plsc.subcore_barrier()

        for b in range(NB):
            pltpu.async_copy(ys.at[isrc.at[b]], bufs[b], gsems[b])

        def body(g, carry):
            base = g * NB
            for b in range(NB):
                t = base + b
                pltpu.make_async_copy(ys.at[isrc.at[t]], bufs[b],
                                      gsems[b]).wait()
                pltpu.sync_copy(bufs[b], acc.at[idst.at[t]], add=True)

                @pl.when(t + NB < TS)
                def _():
                    pltpu.async_copy(ys.at[isrc.at[t + NB]], bufs[b],
                                     gsems[b])
            return carry

        lax.fori_loop(0, NG, body, 0)
        for b in range(NT):
            t = NG * NB + b
            pltpu.make_async_copy(ys.at[isrc.at[t]], bufs[b],
                                  gsems[b]).wait()
            pltpu.sync_copy(bufs[b], acc.at[idst.at[t]], add=True)
        plsc.subcore_barrier()
        pltpu.sync_copy(acc.at[pl.ds(s * STRIPE, STRIPE)],
                        out.at[c, pl.ds(s * STRIPE, STRIPE)])

    return prop


_sc_prop48 = _make_prop(F1)
_sc_prop16 = _make_prop(F2)


def _tc0_body(x_ref, w1r_ref, b1_ref, w1i_ref, root1_ref, xw1i_ref):
    xb = x_ref[...]
    root1_ref[...] = (
        jnp.dot(xb, w1r_ref[...], preferred_element_type=_f32) + b1_ref[...])
    xw1i_ref[...] = jnp.dot(xb, w1i_ref[...], preferred_element_type=_f32)


_tc0 = pl.pallas_call(
    _tc0_body,
    grid=GRID,
    in_specs=[
        pl.BlockSpec((BLK, 128), lambda i: (i, 0)),
        pl.BlockSpec((128, F1), lambda i: (0, 0)),
        pl.BlockSpec((1, F1), lambda i: (0, 0)),
        pl.BlockSpec((128, F1), lambda i: (0, 0)),
    ],
    out_specs=[
        pl.BlockSpec((BLK, F1), lambda i: (i, 0)),
        pl.BlockSpec((BLK, F1), lambda i: (i, 0)),
    ],
    out_shape=[
        jax.ShapeDtypeStruct((NP, F1), _f32),
        jax.ShapeDtypeStruct((NP, F1), _f32),
    ],
)


def _tc1_body(degp_ref, xw1i_ref, dinv_ref, y0s_ref):
    deg = jnp.sum(degp_ref[...], axis=0)
    dinv = jnp.where(deg > 0, lax.rsqrt(deg), 0.0)[:, None]
    dinv_ref[...] = dinv
    y0s_ref[...] = dinv * xw1i_ref[...]


_tc1 = pl.pallas_call(
    _tc1_body,
    grid=GRID,
    in_specs=[
        pl.BlockSpec((NW, BLK), lambda i: (0, i)),
        pl.BlockSpec((BLK, F1), lambda i: (i, 0)),
    ],
    out_specs=[
        pl.BlockSpec((BLK, 1), lambda i: (i, 0)),
        pl.BlockSpec((BLK, F1), lambda i: (i, 0)),
    ],
    out_shape=[
        jax.ShapeDtypeStruct((NP, 1), _f32),
        jax.ShapeDtypeStruct((NP, F1), _f32),
    ],
)


def _tc2_body(agg_ref, dinv_ref, root1_ref, wh_ref, y1s_ref):
    a = agg_ref[0] + agg_ref[1]
    out0 = jnp.maximum(dinv_ref[...] * a + root1_ref[...], 0.0)
    y1s_ref[...] = dinv_ref[...] * jnp.dot(
        out0, wh_ref[...], preferred_element_type=_f32)


_tc2 = pl.pallas_call(
    _tc2_body,
    grid=GRID,
    in_specs=[
        pl.BlockSpec((2, BLK, F1), lambda i: (0, i, 0)),
        pl.BlockSpec((BLK, 1), lambda i: (i, 0)),
        pl.BlockSpec((BLK, F1), lambda i: (i, 0)),
        pl.BlockSpec((F1, F1), lambda i: (0, 0)),
    ],
    out_specs=pl.BlockSpec((BLK, F1), lambda i: (i, 0)),
    out_shape=jax.ShapeDtypeStruct((NP, F1), _f32),
)


def _tc3_body(agg_ref, dinv_ref, root1_ref, w2r_ref, b2_ref, w2i_ref,
              root2_ref, z0s_ref):
    a = agg_ref[0] + agg_ref[1]
    out1 = jnp.maximum(dinv_ref[...] * a + root1_ref[...], 0.0)
    h = (out1[:, 0:16] + out1[:, 16:32] + out1[:, 32:48]) * (1.0 / 3.0)
    h = jnp.maximum(h, 0.0)
    root2_ref[...] = (
        jnp.dot(h, w2r_ref[...], preferred_element_type=_f32) + b2_ref[...])
    z0s_ref[...] = dinv_ref[...] * jnp.dot(
        h, w2i_ref[...], preferred_element_type=_f32)


_tc3 = pl.pallas_call(
    _tc3_body,
    grid=GRID,
    in_specs=[
        pl.BlockSpec((2, BLK, F1), lambda i: (0, i, 0)),
        pl.BlockSpec((BLK, 1), lambda i: (i, 0)),
        pl.BlockSpec((BLK, F1), lambda i: (i, 0)),
        pl.BlockSpec((H1, F2), lambda i: (0, 0)),
        pl.BlockSpec((1, F2), lambda i: (0, 0)),
        pl.BlockSpec((H1, F2), lambda i: (0, 0)),
    ],
    out_specs=[
        pl.BlockSpec((BLK, F2), lambda i: (i, 0)),
        pl.BlockSpec((BLK, F2), lambda i: (i, 0)),
    ],
    out_shape=[
        jax.ShapeDtypeStruct((NP, F2), _f32),
        jax.ShapeDtypeStruct((NP, F2), _f32),
    ],
)


def _tc4_body(agg_ref, dinv_ref, root2_ref, wh_ref, z1s_ref):
    a = agg_ref[0] + agg_ref[1]
    out2 = dinv_ref[...] * a + root2_ref[...]
    z1s_ref[...] = dinv_ref[...] * jnp.dot(
        out2, wh_ref[...], preferred_element_type=_f32)


_tc4 = pl.pallas_call(
    _tc4_body,
    grid=GRID,
    in_specs=[
        pl.BlockSpec((2, BLK, F2), lambda i: (0, i, 0)),
        pl.BlockSpec((BLK, 1), lambda i: (i, 0)),
        pl.BlockSpec((BLK, F2), lambda i: (i, 0)),
        pl.BlockSpec((F2, F2), lambda i: (0, 0)),
    ],
    out_specs=pl.BlockSpec((BLK, F2), lambda i: (i, 0)),
    out_shape=jax.ShapeDtypeStruct((NP, F2), _f32),
)


def _tc5_body(agg_ref, dinv_ref, root2_ref, out_ref):
    a = agg_ref[0] + agg_ref[1]
    o3 = dinv_ref[...] * a + root2_ref[...]
    o = (o3[:, 0:2] + o3[:, 2:4] + o3[:, 4:6]) * (1.0 / 3.0)
    m = jnp.max(o, axis=1, keepdims=True)
    lse = m + jnp.log(jnp.sum(jnp.exp(o - m), axis=1, keepdims=True))
    out_ref[...] = o - lse


_tc5 = pl.pallas_call(
    _tc5_body,
    grid=GRID,
    in_specs=[
        pl.BlockSpec((2, BLK, F2), lambda i: (0, i, 0)),
        pl.BlockSpec((BLK, 1), lambda i: (i, 0)),
        pl.BlockSpec((BLK, F2), lambda i: (i, 0)),
    ],
    out_specs=pl.BlockSpec((BLK, OUT), lambda i: (i, 0)),
    out_shape=jax.ShapeDtypeStruct((NP, OUT), _f32),
)


def kernel(x, edge_index, w1_init, w1_hop, w1_root, b1,
           w2_init, w2_hop, w2_root, b2):
    x_pad = jnp.zeros((NP, 128), _f32).at[:N, :FEA].set(x)
    padi = jnp.full((E2 - E,), N, jnp.int32)
    src_f = jnp.concatenate([edge_index[0], padi]).reshape(NW, EPT)
    dst_f = jnp.concatenate([edge_index[1], padi]).reshape(NW, EPT)
    src3 = src_f.reshape(NW, NCH // SLAB, SLAB * CH)
    dst3 = dst_f.reshape(NW, NCH // SLAB, SLAB * CH)

    w1r = jnp.zeros((128, F1), _f32).at[:FEA].set(
        w1_root.transpose(1, 0, 2).reshape(FEA, F1))
    w1i = jnp.zeros((128, F1), _f32).at[:FEA].set(
        w1_init.transpose(1, 0, 2).reshape(FEA, F1))
    b1f = b1.reshape(1, F1)
    w1h = jax.scipy.linalg.block_diag(w1_hop[0], w1_hop[1], w1_hop[2])

    kout = K * OUT
    w2r = jnp.zeros((H1, F2), _f32).at[:, :kout].set(
        w2_root.transpose(1, 0, 2).reshape(H1, kout))
    w2i = jnp.zeros((H1, F2), _f32).at[:, :kout].set(
        w2_init.transpose(1, 0, 2).reshape(H1, kout))
    b2f = jnp.zeros((1, F2), _f32).at[:, :kout].set(b2.reshape(1, kout))
    w2h = jnp.zeros((F2, F2), _f32).at[:kout, :kout].set(
        jax.scipy.linalg.block_diag(w2_hop[0], w2_hop[1], w2_hop[2]))

    zeros_np = jnp.zeros((NP,), _f32)
    zeros48 = jnp.zeros((NP, F1), _f32)
    zeros16 = jnp.zeros((NP, F2), _f32)

    deg_parts = _sc_degree(dst_f, zeros_np)
    root1, xw1i = _tc0(x_pad, w1r, b1f, w1i)
    dinv, y0s = _tc1(deg_parts, xw1i)
    agg0 = _sc_prop48(y0s, src3, dst3, zeros48)
    y1s = _tc2(agg0, dinv, root1, w1h)
    agg1 = _sc_prop48(y1s, src3, dst3, zeros48)
    root2, z0s = _tc3(agg1, dinv, root1, w2r, b2f, w2i)
    agg2 = _sc_prop16(z0s, src3, dst3, zeros16)
    z1s = _tc4(agg2, dinv, root2, w2h)
    agg3 = _sc_prop16(z1s, src3, dst3, zeros16)
    out = _tc5(agg3, dinv, root2)
    return out[:N]
